# baseline (speedup 1.0000x reference)
import functools

import jax
import jax.numpy as jnp
from jax import lax
from jax.experimental import pallas as pl
from jax.experimental.pallas import tpu as pltpu
from jax.experimental.pallas import tpu_sc as plsc

N = 10000
E = 320000
D = 128

NC = 2
NS = 16
NW = NC * NS
CH = 120
NCH = 84
EPW = NCH * CH
E_PAD = NW * EPW
BLK = 6
NB = NCH // BLK
RPT = 640
ACC = NS * RPT

_mesh = plsc.VectorSubcoreMesh(core_axis_name="c", subcore_axis_name="s")


@functools.partial(
    pl.kernel, mesh=_mesh,
    out_type=jax.ShapeDtypeStruct((NW, ACC), jnp.float32),
    scratch_types=[
        pltpu.VMEM((EPW,), jnp.int32),
        pltpu.VMEM((ACC,), jnp.float32),
    ],
    compiler_params=pltpu.CompilerParams(needs_layout_passes=False),
)
def _deg_kernel(dst_hbm, out_hbm, idxs, hist):
    c = lax.axis_index("c")
    s = lax.axis_index("s")
    w = c * NS + s
    ones = jnp.ones((16,), jnp.float32)

    def _zero(i, carry):
        hist[pl.ds(i * 16, 16)] = jnp.zeros((16,), jnp.float32)
        return carry

    lax.fori_loop(0, ACC // 16, _zero, 0)
    pltpu.sync_copy(dst_hbm.at[pl.ds(w * EPW, EPW)], idxs)

    def _body(i, carry):
        idx = idxs[pl.ds(i * 16, 16)]
        plsc.addupdate_scatter(hist, [idx], ones)
        return carry

    lax.fori_loop(0, EPW // 16, _body, 0)
    pltpu.sync_copy(hist, out_hbm.at[w])


@functools.partial(
    pl.kernel, mesh=_mesh,
    out_type=jax.ShapeDtypeStruct((NC, ACC, D), jnp.float32),
    scratch_types=[
        pltpu.VMEM((BLK * CH,), jnp.int32),
        pltpu.VMEM((BLK, 1, CH), jnp.int32),
        pltpu.VMEM((3, CH, D), jnp.float32),
        pltpu.VMEM_SHARED((ACC, D), jnp.float32),
        pltpu.SemaphoreType.DMA,
        pltpu.SemaphoreType.DMA,
        pltpu.SemaphoreType.DMA,
        pltpu.SemaphoreType.DMA,
        pltpu.SemaphoreType.DMA,
        pltpu.SemaphoreType.DMA,
    ],
)
def _scat_kernel(hn_hbm, src_hbm, dst_hbm, out_hbm, srcb, dstb, rows, acc,
                 gsem0, gsem1, gsem2, ssem0, ssem1, ssem2):
    c = lax.axis_index("c")
    s = lax.axis_index("s")
    w = c * NS + s
    gsems = (gsem0, gsem1, gsem2)
    ssems = (ssem0, ssem1, ssem2)

    def _zero(r, carry):
        for j in range(D // 16):
            rows[0, r, pl.ds(j * 16, 16)] = jnp.zeros((16,), jnp.float32)
        return carry

    lax.fori_loop(0, CH, _zero, 0)
    for z in range(RPT // CH):
        pltpu.sync_copy(rows.at[0], acc.at[pl.ds(s * RPT + z * CH, CH)])
    pltpu.sync_copy(rows.at[0, pl.ds(0, RPT - (RPT // CH) * CH)],
                    acc.at[pl.ds(s * RPT + (RPT // CH) * CH,
                                 RPT - (RPT // CH) * CH)])
    plsc.subcore_barrier()

    def _gather(q, b):
        return pltpu.async_copy(
            hn_hbm.at[srcb.at[pl.ds(q * CH, CH)]], rows.at[b], gsems[b])

    def _scatter(q, b):
        return pltpu.async_copy(
            rows.at[b], acc.at[dstb.at[q, 0]], ssems[b], add=True)

    def _body(k, carry):
        pltpu.sync_copy(src_hbm.at[pl.ds(w * EPW + k * BLK * CH, BLK * CH)], srcb)
        pltpu.sync_copy(dst_hbm.at[w, k], dstb)
        g0 = _gather(0, 0)
        g1 = _gather(1, 1)
        g2 = _gather(2, 2)
        g0.wait()
        s0 = _scatter(0, 0)
        g1.wait()
        s1 = _scatter(1, 1)
        g2.wait()
        s2 = _scatter(2, 2)
        s0.wait()
        g3 = _gather(3, 0)
        s1.wait()
        g4 = _gather(4, 1)
        s2.wait()
        g5 = _gather(5, 2)
        g3.wait()
        s3 = _scatter(3, 0)
        g4.wait()
        s4 = _scatter(4, 1)
        g5.wait()
        s5 = _scatter(5, 2)
        s3.wait()
        s4.wait()
        s5.wait()
        return carry

    lax.fori_loop(0, NB, _body, 0)
    plsc.subcore_barrier()
    pltpu.sync_copy(acc.at[pl.ds(s * RPT, RPT)],
                    out_hbm.at[c, pl.ds(s * RPT, RPT)])


MB = 2000
GRID = N // MB


def _mm1_body(x_ref, w_ref, o_ref):
    o_ref[...] = jnp.dot(x_ref[...], w_ref[...],
                         preferred_element_type=jnp.float32)


_mm1 = pl.pallas_call(
    _mm1_body,
    grid=(GRID,),
    in_specs=[
        pl.BlockSpec((MB, D), lambda i: (i, 0)),
        pl.BlockSpec((D, D), lambda i: (0, 0)),
    ],
    out_specs=pl.BlockSpec((MB, D), lambda i: (i, 0)),
    out_shape=jax.ShapeDtypeStruct((N, D), jnp.float32),
)


def _scale_body(h_ref, hist_ref, o_ref):
    dinv = lax.rsqrt(jnp.sum(hist_ref[...], axis=1, keepdims=True) + 1.0)
    o_ref[...] = h_ref[...] * dinv


_scale = pl.pallas_call(
    _scale_body,
    grid=(GRID,),
    in_specs=[
        pl.BlockSpec((MB, D), lambda i: (i, 0)),
        pl.BlockSpec((MB, NW), lambda i: (i, 0)),
    ],
    out_specs=pl.BlockSpec((MB, D), lambda i: (i, 0)),
    out_shape=jax.ShapeDtypeStruct((N, D), jnp.float32),
)


def _mid_body(s1_ref, hn1_ref, hist_ref, b1_ref, w2_ref, o_ref):
    dinv = lax.rsqrt(jnp.sum(hist_ref[...], axis=1, keepdims=True) + 1.0)
    t = dinv * (s1_ref[0] + s1_ref[1] + hn1_ref[...]) + b1_ref[...]
    t = jnp.maximum(t, 0.0)
    o_ref[...] = dinv * jnp.dot(t, w2_ref[...], preferred_element_type=jnp.float32)


_mid = pl.pallas_call(
    _mid_body,
    grid=(GRID,),
    in_specs=[
        pl.BlockSpec((NC, MB, D), lambda i: (0, i, 0)),
        pl.BlockSpec((MB, D), lambda i: (i, 0)),
        pl.BlockSpec((MB, NW), lambda i: (i, 0)),
        pl.BlockSpec((1, D), lambda i: (0, 0)),
        pl.BlockSpec((D, D), lambda i: (0, 0)),
    ],
    out_specs=pl.BlockSpec((MB, D), lambda i: (i, 0)),
    out_shape=jax.ShapeDtypeStruct((N, D), jnp.float32),
)


def _fin_body(s2_ref, hn2_ref, hist_ref, b2_ref, o_ref):
    dinv = lax.rsqrt(jnp.sum(hist_ref[...], axis=1, keepdims=True) + 1.0)
    o_ref[...] = dinv * (s2_ref[0] + s2_ref[1] + hn2_ref[...]) + b2_ref[...]


_fin = pl.pallas_call(
    _fin_body,
    grid=(GRID,),
    in_specs=[
        pl.BlockSpec((NC, MB, D), lambda i: (0, i, 0)),
        pl.BlockSpec((MB, D), lambda i: (i, 0)),
        pl.BlockSpec((MB, NW), lambda i: (i, 0)),
        pl.BlockSpec((1, D), lambda i: (0, 0)),
    ],
    out_specs=pl.BlockSpec((MB, D), lambda i: (i, 0)),
    out_shape=jax.ShapeDtypeStruct((N, D), jnp.float32),
)


def kernel(x, edge_index, W1, b1, W2, b2):
    ei = edge_index.astype(jnp.int32)
    pad = E_PAD - E
    pad_dst = N + (jnp.arange(pad, dtype=jnp.int32) % (ACC - N))
    pad_src = jnp.arange(pad, dtype=jnp.int32) % N
    src_p = jnp.concatenate([ei[0], pad_src])
    dst_p = jnp.concatenate([ei[1], pad_dst])
    dst_w4 = dst_p.reshape(NW, NB, BLK, 1, CH)
    b1r = b1.reshape(1, D)
    b2r = b2.reshape(1, D)

    h1 = _mm1(x, W1)
    hist = _deg_kernel(dst_p).T
    hn1 = _scale(h1, hist)
    s1 = _scat_kernel(hn1, src_p, dst_w4)
    hn2 = _mid(s1, hn1, hist, b1r, W2)
    s2 = _scat_kernel(hn2, src_p, dst_w4)
    return _fin(s2, hn2, hist, b2r)


# speedup vs baseline: 1.0970x; 1.0970x over previous
import functools

import jax
import jax.numpy as jnp
from jax import lax
from jax.experimental import pallas as pl
from jax.experimental.pallas import tpu as pltpu
from jax.experimental.pallas import tpu_sc as plsc

N = 10000
E = 320000
D = 128

NC = 2
NS = 16
NW = NC * NS
CH = 120
NCH = 84
EPW = NCH * CH
E_PAD = NW * EPW
BLK = 12
NB = NCH // BLK
RPT = 640
ACC = NS * RPT

_mesh = plsc.VectorSubcoreMesh(core_axis_name="c", subcore_axis_name="s")


@functools.partial(
    pl.kernel, mesh=_mesh,
    out_type=jax.ShapeDtypeStruct((NW, ACC), jnp.float32),
    scratch_types=[
        pltpu.VMEM((EPW,), jnp.int32),
        pltpu.VMEM((ACC,), jnp.float32),
    ],
    compiler_params=pltpu.CompilerParams(needs_layout_passes=False),
)
def _deg_kernel(dst_hbm, out_hbm, idxs, hist):
    c = lax.axis_index("c")
    s = lax.axis_index("s")
    w = c * NS + s
    ones = jnp.ones((16,), jnp.float32)

    def _zero(i, carry):
        hist[pl.ds(i * 16, 16)] = jnp.zeros((16,), jnp.float32)
        return carry

    lax.fori_loop(0, ACC // 16, _zero, 0)
    pltpu.sync_copy(dst_hbm.at[pl.ds(w * EPW, EPW)], idxs)

    def _body(i, carry):
        idx = idxs[pl.ds(i * 16, 16)]
        plsc.addupdate_scatter(hist, [idx], ones)
        return carry

    lax.fori_loop(0, EPW // 16, _body, 0)
    pltpu.sync_copy(hist, out_hbm.at[w])


@functools.partial(
    pl.kernel, mesh=_mesh,
    out_type=jax.ShapeDtypeStruct((NC, ACC, D), jnp.float32),
    scratch_types=[
        pltpu.VMEM((BLK * CH,), jnp.int32),
        pltpu.VMEM((BLK, 1, CH), jnp.int32),
        pltpu.VMEM((3, CH, D), jnp.float32),
        pltpu.VMEM_SHARED((ACC, D), jnp.float32),
        pltpu.SemaphoreType.DMA,
        pltpu.SemaphoreType.DMA,
        pltpu.SemaphoreType.DMA,
        pltpu.SemaphoreType.DMA,
        pltpu.SemaphoreType.DMA,
        pltpu.SemaphoreType.DMA,
    ],
)
def _scat_kernel(hn_hbm, src_hbm, dst_hbm, out_hbm, srcb, dstb, rows, acc,
                 gsem0, gsem1, gsem2, ssem0, ssem1, ssem2):
    c = lax.axis_index("c")
    s = lax.axis_index("s")
    w = c * NS + s
    gsems = (gsem0, gsem1, gsem2)
    ssems = (ssem0, ssem1, ssem2)

    def _zero(r, carry):
        for j in range(D // 16):
            rows[0, r, pl.ds(j * 16, 16)] = jnp.zeros((16,), jnp.float32)
        return carry

    lax.fori_loop(0, CH, _zero, 0)
    for z in range(RPT // CH):
        pltpu.sync_copy(rows.at[0], acc.at[pl.ds(s * RPT + z * CH, CH)])
    pltpu.sync_copy(rows.at[0, pl.ds(0, RPT - (RPT // CH) * CH)],
                    acc.at[pl.ds(s * RPT + (RPT // CH) * CH,
                                 RPT - (RPT // CH) * CH)])
    plsc.subcore_barrier()

    def _gather(q, b):
        return pltpu.async_copy(
            hn_hbm.at[srcb.at[pl.ds(q * CH, CH)]], rows.at[b], gsems[b])

    def _scatter(q, b):
        return pltpu.async_copy(
            rows.at[b], acc.at[dstb.at[q, 0]], ssems[b], add=True)

    def _body(k, carry):
        pltpu.sync_copy(src_hbm.at[pl.ds(w * EPW + k * BLK * CH, BLK * CH)], srcb)
        pltpu.sync_copy(dst_hbm.at[w, k], dstb)
        gs = [_gather(q, q) for q in range(3)]
        for r in range(BLK // 3):
            base = r * 3
            ss = []
            for j in range(3):
                gs[j].wait()
                ss.append(_scatter(base + j, j))
            for j in range(3):
                ss[j].wait()
                if base + 3 + j < BLK:
                    gs[j] = _gather(base + 3 + j, j)
        return carry

    lax.fori_loop(0, NB, _body, 0)
    plsc.subcore_barrier()
    pltpu.sync_copy(acc.at[pl.ds(s * RPT, RPT)],
                    out_hbm.at[c, pl.ds(s * RPT, RPT)])


MB = 2000
GRID = N // MB


def _lin1_body(x_ref, w_ref, hist_ref, o_ref):
    dinv = lax.rsqrt(jnp.sum(hist_ref[...], axis=1, keepdims=True) + 1.0)
    h = jnp.dot(x_ref[...], w_ref[...], preferred_element_type=jnp.float32)
    o_ref[...] = h * dinv


_lin1 = pl.pallas_call(
    _lin1_body,
    grid=(GRID,),
    in_specs=[
        pl.BlockSpec((MB, D), lambda i: (i, 0)),
        pl.BlockSpec((D, D), lambda i: (0, 0)),
        pl.BlockSpec((MB, NW), lambda i: (i, 0)),
    ],
    out_specs=pl.BlockSpec((MB, D), lambda i: (i, 0)),
    out_shape=jax.ShapeDtypeStruct((N, D), jnp.float32),
)


def _mid_body(s1_ref, hn1_ref, hist_ref, b1_ref, w2_ref, o_ref):
    dinv = lax.rsqrt(jnp.sum(hist_ref[...], axis=1, keepdims=True) + 1.0)
    t = dinv * (s1_ref[0] + s1_ref[1] + hn1_ref[...]) + b1_ref[...]
    t = jnp.maximum(t, 0.0)
    o_ref[...] = dinv * jnp.dot(t, w2_ref[...], preferred_element_type=jnp.float32)


_mid = pl.pallas_call(
    _mid_body,
    grid=(GRID,),
    in_specs=[
        pl.BlockSpec((NC, MB, D), lambda i: (0, i, 0)),
        pl.BlockSpec((MB, D), lambda i: (i, 0)),
        pl.BlockSpec((MB, NW), lambda i: (i, 0)),
        pl.BlockSpec((1, D), lambda i: (0, 0)),
        pl.BlockSpec((D, D), lambda i: (0, 0)),
    ],
    out_specs=pl.BlockSpec((MB, D), lambda i: (i, 0)),
    out_shape=jax.ShapeDtypeStruct((N, D), jnp.float32),
)


def _fin_body(s2_ref, hn2_ref, hist_ref, b2_ref, o_ref):
    dinv = lax.rsqrt(jnp.sum(hist_ref[...], axis=1, keepdims=True) + 1.0)
    o_ref[...] = dinv * (s2_ref[0] + s2_ref[1] + hn2_ref[...]) + b2_ref[...]


_fin = pl.pallas_call(
    _fin_body,
    grid=(GRID,),
    in_specs=[
        pl.BlockSpec((NC, MB, D), lambda i: (0, i, 0)),
        pl.BlockSpec((MB, D), lambda i: (i, 0)),
        pl.BlockSpec((MB, NW), lambda i: (i, 0)),
        pl.BlockSpec((1, D), lambda i: (0, 0)),
    ],
    out_specs=pl.BlockSpec((MB, D), lambda i: (i, 0)),
    out_shape=jax.ShapeDtypeStruct((N, D), jnp.float32),
)


def kernel(x, edge_index, W1, b1, W2, b2):
    ei = edge_index.astype(jnp.int32)
    pad = E_PAD - E
    pad_dst = N + (jnp.arange(pad, dtype=jnp.int32) % (ACC - N))
    pad_src = jnp.arange(pad, dtype=jnp.int32) % N
    src_p = jnp.concatenate([ei[0], pad_src])
    dst_p = jnp.concatenate([ei[1], pad_dst])
    dst_w4 = dst_p.reshape(NW, NB, BLK, 1, CH)
    b1r = b1.reshape(1, D)
    b2r = b2.reshape(1, D)

    hist = _deg_kernel(dst_p).T
    hn1 = _lin1(x, W1, hist)
    s1 = _scat_kernel(hn1, src_p, dst_w4)
    hn2 = _mid(s1, hn1, hist, b1r, W2)
    s2 = _scat_kernel(hn2, src_p, dst_w4)
    return _fin(s2, hn2, hist, b2r)
